# baseline (speedup 1.0000x reference)
import functools
import math

import jax
import jax.numpy as jnp
from jax import lax
from jax.experimental import pallas as pl
from jax.experimental.pallas import tpu as pltpu
from jax.experimental.pallas import tpu_sc as plsc

_NW = 32
_NBUF = 6
_LANES = 16


def _emb_call(B0, B1, V, D):
    rows_per_w = B0 // _NW
    mesh = plsc.VectorSubcoreMesh(core_axis_name="c", subcore_axis_name="s")

    @functools.partial(
        pl.kernel,
        mesh=mesh,
        compiler_params=pltpu.CompilerParams(use_tc_tiling_on_sc=False),
        out_type=jax.ShapeDtypeStruct((B0 * B1, D), jnp.float32),
        scratch_types=[
            pltpu.VMEM((rows_per_w, B1), jnp.int32),
            pltpu.VMEM((_NBUF, B1, D), jnp.float32),
            pltpu.VMEM((_NBUF, B1, D), jnp.float32),
            pltpu.SemaphoreType.DMA((_NBUF,)),
            pltpu.SemaphoreType.DMA((_NBUF,)),
        ],
    )
    def emb_kernel(x_hbm, lut_hbm, out_hbm, idx_v, rows_v, out_v, in_sem, out_sem):
        scale = jnp.float32(math.sqrt(D))
        wid = lax.axis_index("s") * 2 + lax.axis_index("c")
        base = wid * rows_per_w
        pltpu.sync_copy(x_hbm.at[pl.ds(base, rows_per_w)], idx_v)

        def gather(j, b):
            pltpu.async_copy(lut_hbm.at[idx_v.at[j]], rows_v.at[b], in_sem.at[b])

        def wait_in(b):
            pltpu.make_async_copy(
                lut_hbm.at[idx_v.at[0]], rows_v.at[b], in_sem.at[b]
            ).wait()

        def put(j, b):
            pltpu.async_copy(
                out_v.at[b], out_hbm.at[pl.ds((base + j) * B1, B1)], out_sem.at[b]
            )

        def wait_out(b):
            pltpu.make_async_copy(
                out_v.at[b], out_hbm.at[pl.ds(base * B1, B1)], out_sem.at[b]
            ).wait()

        for b in range(_NBUF):
            gather(b, b)

        def step(j, b):
            wait_in(b)

            @pl.when(j >= _NBUF)
            def _():
                wait_out(b)

            def srow(r, _):
                for c in range(D // _LANES):
                    sl = pl.ds(c * _LANES, _LANES)
                    out_v[b, r, sl] = rows_v[b, r, sl] * scale
                return 0

            lax.fori_loop(0, B1, srow, 0, unroll=2)

            @pl.when(j + _NBUF < rows_per_w)
            def _():
                gather(j + _NBUF, b)

            put(j, b)

        def outer(i, _):
            j0 = i * _NBUF
            for b in range(_NBUF):
                step(j0 + b, b)
            return 0

        n_full = rows_per_w // _NBUF
        lax.fori_loop(0, n_full, outer, 0)
        for t in range(rows_per_w - n_full * _NBUF):
            step(n_full * _NBUF + t, t)
        for b in range(_NBUF):
            wait_out(b)

    return emb_kernel


def kernel(x, lut):
    B0, B1 = x.shape
    V, D = lut.shape
    out = _emb_call(B0, B1, V, D)(x.astype(jnp.int32), lut)
    return out.reshape(B0, B1, D)


# speedup vs baseline: 1.3441x; 1.3441x over previous
import functools
import math

import jax
import jax.numpy as jnp
from jax import lax
from jax.experimental import pallas as pl
from jax.experimental.pallas import tpu as pltpu
from jax.experimental.pallas import tpu_sc as plsc

_NW = 32
_G = 128
_NBUF = 5
_LANES = 16


def _emb_call(B, V, D):
    b_per_w = B // _NW
    n_g = b_per_w // _G
    mesh = plsc.VectorSubcoreMesh(core_axis_name="c", subcore_axis_name="s")

    @functools.partial(
        pl.kernel,
        mesh=mesh,
        compiler_params=pltpu.CompilerParams(use_tc_tiling_on_sc=False),
        out_type=jax.ShapeDtypeStruct((B, D), jnp.float32),
        scratch_types=[
            pltpu.VMEM((b_per_w,), jnp.int32),
            pltpu.VMEM((_NBUF, _G, D), jnp.float32),
            pltpu.SemaphoreType.DMA((_NBUF,)),
            pltpu.SemaphoreType.DMA((_NBUF,)),
        ],
    )
    def emb_kernel(x_hbm, lut_hbm, out_hbm, idx_v, rows_v, in_sem, out_sem):
        scale = jnp.float32(math.sqrt(D))
        wid = lax.axis_index("s") * 2 + lax.axis_index("c")
        base = wid * b_per_w
        pltpu.sync_copy(x_hbm.at[pl.ds(base, b_per_w)], idx_v)

        def gather(g, b):
            pltpu.async_copy(
                lut_hbm.at[idx_v.at[pl.ds(g * _G, _G)]], rows_v.at[b], in_sem.at[b]
            )

        def wait_in(b):
            pltpu.make_async_copy(
                lut_hbm.at[idx_v.at[pl.ds(0, _G)]], rows_v.at[b], in_sem.at[b]
            ).wait()

        def put(g, b):
            pltpu.async_copy(
                rows_v.at[b], out_hbm.at[pl.ds(base + g * _G, _G)], out_sem.at[b]
            )

        def wait_out(b):
            pltpu.make_async_copy(
                rows_v.at[b], out_hbm.at[pl.ds(base, _G)], out_sem.at[b]
            ).wait()

        for b in range(_NBUF):
            gather(b, b)

        def outer(i, _):
            g0 = i * _NBUF
            for b in range(_NBUF):
                g = g0 + b
                wait_in(b)

                def srow(r, _):
                    for c in range(D // _LANES):
                        sl = pl.ds(c * _LANES, _LANES)
                        rows_v[b, r, sl] = rows_v[b, r, sl] * scale
                    return 0

                lax.fori_loop(0, _G, srow, 0, unroll=2)
                put(g, b)

                @pl.when(g + _NBUF < n_g)
                def _():
                    wait_out(b)
                    gather(g + _NBUF, b)

            return 0

        lax.fori_loop(0, n_g // _NBUF, outer, 0)
        for b in range(_NBUF):
            wait_out(b)

    return emb_kernel


def kernel(x, lut):
    B0, B1 = x.shape
    V, D = lut.shape
    B = B0 * B1
    x_flat = x.reshape(B).astype(jnp.int32)
    out = _emb_call(B, V, D)(x_flat, lut)
    return out.reshape(B0, B1, D)
